# baseline (speedup 1.0000x reference)
import functools

import jax
import jax.numpy as jnp
from jax import lax
from jax.experimental import pallas as pl
from jax.experimental.pallas import tpu as pltpu
from jax.experimental.pallas import tpu_sc as plsc

N, E, D, H = 10000, 320000, 128, 128
NC, NS = 2, 16
NW = NC * NS
CHUNK = 128
ROWS2 = 2560
E_PAD = ROWS2 * CHUNK
HK = 2
ROWS_H = ROWS2 // HK
E_PADH = ROWS_H * CHUNK
E_H = E // HK
CPW = ROWS_H // NW
PAIRS = CPW // 4
HPAIRS = CPW // 2
N_PAD = 10240
NPT = N_PAD // NS

_mesh = functools.partial(
    plsc.VectorSubcoreMesh,
    core_axis_name="c", subcore_axis_name="s", num_cores=NC, num_subcores=NS,
)


def _wid():
    return (lax.axis_index("s") * NC + lax.axis_index("c")).astype(jnp.int32)


def _gather_body(ps_hbm, pr_hbm, sidx_hbm, ridx_hbm, gs_hbm, gr_hbm,
                 idx_v, b0, b1, gsem, osem0, osem1):
    wid = _wid()
    base_c = wid * CPW
    for tab, idx_hbm, out in ((ps_hbm, sidx_hbm, gs_hbm),
                              (pr_hbm, ridx_hbm, gr_hbm)):
        pltpu.sync_copy(idx_hbm.at[pl.ds(base_c, CPW)], idx_v)

        def pair(p, carry, tab=tab, out=out):
            c0 = p * 4
            for buf, osem, cb in ((b0, osem0, c0), (b1, osem1, c0 + 2)):
                dst = out.at[pl.ds((base_c + cb) * CHUNK, 2 * CHUNK)]

                @pl.when(p > 0)
                def _(buf=buf, osem=osem, dst=dst):
                    pltpu.make_async_copy(dst, buf, osem).wait()

                d0 = pltpu.async_copy(tab.at[idx_v.at[cb]],
                                      buf.at[pl.ds(0, CHUNK)], gsem)
                d1 = pltpu.async_copy(tab.at[idx_v.at[cb + 1]],
                                      buf.at[pl.ds(CHUNK, CHUNK)], gsem)
                d0.wait()
                d1.wait()
                pltpu.async_copy(buf, dst, osem)
            return carry

        lax.fori_loop(0, PAIRS, pair, 0)
        pltpu.make_async_copy(out.at[pl.ds(0, 2 * CHUNK)], b0, osem0).wait()
        pltpu.make_async_copy(out.at[pl.ds(0, 2 * CHUNK)], b1, osem1).wait()


def _scatter_body(u_hbm, ridx_hbm, out_hbm, idx_v, ub0, ub1,
                  agg_sh, lsem0, lsem1):
    c = lax.axis_index("c").astype(jnp.int32)
    s = lax.axis_index("s").astype(jnp.int32)
    wid = s * NC + c
    zero = jnp.zeros((16,), jnp.float32)
    zb = 32
    for i in range(zb):
        for l in range(D // 16):
            ub0[i, pl.ds(l * 16, 16)] = zero
    zsrc = ub0.at[pl.ds(0, zb)]
    for k in range(NPT // zb):
        pltpu.sync_copy(zsrc, agg_sh.at[pl.ds(s * NPT + k * zb, zb)])
    plsc.subcore_barrier()
    base_c = wid * CPW
    pltpu.sync_copy(ridx_hbm.at[pl.ds(base_c, CPW)], idx_v)
    pltpu.async_copy(u_hbm.at[pl.ds(base_c * CHUNK, CHUNK)], ub0, lsem0)
    pltpu.async_copy(u_hbm.at[pl.ds((base_c + 1) * CHUNK, CHUNK)], ub1, lsem1)

    def pair(p, carry):
        for off, buf, lsem in ((0, ub0, lsem0), (1, ub1, lsem1)):
            cb = 2 * p + off
            src = u_hbm.at[pl.ds((base_c + cb) * CHUNK, CHUNK)]
            pltpu.make_async_copy(src, buf, lsem).wait()
            pltpu.sync_copy(buf, agg_sh.at[idx_v.at[cb]], add=True)

            @pl.when(p < HPAIRS - 1)
            def _(buf=buf, lsem=lsem, cb=cb):
                pltpu.async_copy(
                    u_hbm.at[pl.ds((base_c + cb + 2) * CHUNK, CHUNK)],
                    buf, lsem)
        return carry

    lax.fori_loop(0, HPAIRS, pair, 0)
    plsc.subcore_barrier()
    ob = ub0.shape[0]
    for k in range(NPT // ob):
        off = s * NPT + k * ob
        pltpu.sync_copy(agg_sh.at[pl.ds(off, ob)], ub0)
        pltpu.sync_copy(ub0, out_hbm.at[c].at[pl.ds(off, ob)])


def _proj_body(node_ref, ws_ref, wr_ref, ps_ref, pr_ref):
    x = node_ref[...]
    ps_ref[...] = jnp.dot(x, ws_ref[...], preferred_element_type=jnp.float32)
    pr_ref[...] = jnp.dot(x, wr_ref[...], preferred_element_type=jnp.float32)


def _edge_body(gs_ref, gr_ref, ef_ref, we_ref, we2_ref, b1_ref, b2_ref,
               u_ref, ne_ref):
    ef = ef_ref[...]
    x = gs_ref[...] + gr_ref[...] + b1_ref[...]
    x = x + jnp.dot(ef, we_ref[...], preferred_element_type=jnp.float32)
    h = jnp.maximum(x, 0.0)
    u = jnp.dot(h, we2_ref[...], preferred_element_type=jnp.float32) + b2_ref[...]
    u_ref[...] = u
    ne_ref[...] = u + ef


def _edge_body_alias(gs_ref, gr_ref, ef_ref, we_ref, we2_ref, b1_ref, b2_ref,
                     ne_prev_ref, u_ref, ne_ref):
    del ne_prev_ref
    _edge_body(gs_ref, gr_ref, ef_ref, we_ref, we2_ref, b1_ref, b2_ref,
               u_ref, ne_ref)


def _node_body(nf_ref, p0_ref, p1_ref, p2_ref, p3_ref,
               w1a_ref, w1b_ref, w2_ref, b1_ref, b2_ref, out_ref):
    nf = nf_ref[...]
    agg = (p0_ref[...] + p1_ref[...]) + (p2_ref[...] + p3_ref[...])
    x = (jnp.dot(nf, w1a_ref[...], preferred_element_type=jnp.float32)
         + jnp.dot(agg, w1b_ref[...], preferred_element_type=jnp.float32)
         + b1_ref[...])
    h = jnp.maximum(x, 0.0)
    out_ref[...] = (jnp.dot(h, w2_ref[...], preferred_element_type=jnp.float32)
                    + b2_ref[...] + nf)


_BN = 1024
_BNM = 1000
_BE = 1280


def _full(i):
    return (0, 0)


def _rowblk(i):
    return (i, 0)


def kernel(node_features, edge_features, senders, receivers,
           We1, be1, We2, be2, Wn1, bn1, Wn2, bn2):
    f32 = jnp.float32
    pad_e = E_PAD - E
    ar = jnp.arange(pad_e, dtype=jnp.int32)
    sidx = jnp.concatenate([senders.astype(jnp.int32),
                            ar % N_PAD]).reshape(ROWS2, CHUNK)
    ridx = jnp.concatenate([receivers.astype(jnp.int32),
                            N + ar % (N_PAD - N)]).reshape(ROWS2, CHUNK)
    Ws, Wr, We = We1[:D], We1[D:2 * D], We1[2 * D:]
    Wn1a, Wn1b = Wn1[:D], Wn1[D:]
    b_e1, b_e2 = be1.reshape(1, H), be2.reshape(1, D)
    b_n1, b_n2 = bn1.reshape(1, H), bn2.reshape(1, D)

    ps, pr = pl.pallas_call(
        _proj_body,
        grid=(N_PAD // _BN,),
        in_specs=[pl.BlockSpec((_BN, D), _rowblk),
                  pl.BlockSpec((D, D), _full),
                  pl.BlockSpec((D, D), _full)],
        out_specs=[pl.BlockSpec((_BN, D), _rowblk)] * 2,
        out_shape=[jax.ShapeDtypeStruct((N_PAD, D), f32)] * 2,
    )(node_features, Ws, Wr)

    gather = pl.kernel(
        _gather_body,
        out_type=[jax.ShapeDtypeStruct((E_PADH, D), f32)] * 2,
        mesh=_mesh(),
        scratch_types=[pltpu.VMEM((CPW, CHUNK), jnp.int32),
                       pltpu.VMEM((2 * CHUNK, D), f32),
                       pltpu.VMEM((2 * CHUNK, D), f32),
                       pltpu.SemaphoreType.DMA,
                       pltpu.SemaphoreType.DMA,
                       pltpu.SemaphoreType.DMA],
    )
    scatter = pl.kernel(
        _scatter_body,
        out_type=jax.ShapeDtypeStruct((NC, N_PAD, D), f32),
        mesh=_mesh(),
        scratch_types=[pltpu.VMEM((CPW, CHUNK), jnp.int32),
                       pltpu.VMEM((CHUNK, D), f32),
                       pltpu.VMEM((CHUNK, D), f32),
                       pltpu.VMEM_SHARED((N_PAD, D), f32),
                       pltpu.SemaphoreType.DMA,
                       pltpu.SemaphoreType.DMA],
    )

    real_rows = [E_PADH] * (HK - 1) + [E - (HK - 1) * E_PADH]
    blk_off = [h * (E_PADH // _BE) for h in range(HK)]
    new_edge = None
    parts = []
    gpairs = [gather(ps, pr, sidx[h * ROWS_H:(h + 1) * ROWS_H],
                     ridx[h * ROWS_H:(h + 1) * ROWS_H]) for h in range(HK)]
    for h in range(HK):
        gs_h, gr_h = gpairs[h]
        blks_h = real_rows[h] // _BE

        def _ef_blk(i, h=h):
            return (i + blk_off[h], 0)

        ins = [gs_h, gr_h, edge_features, We, We2, b_e1, b_e2]
        in_specs = [pl.BlockSpec((_BE, D), _rowblk),
                    pl.BlockSpec((_BE, D), _rowblk),
                    pl.BlockSpec((_BE, D), _ef_blk),
                    pl.BlockSpec((D, H), _full),
                    pl.BlockSpec((H, D), _full),
                    pl.BlockSpec((1, H), _full),
                    pl.BlockSpec((1, D), _full)]
        body = _edge_body
        io_aliases = {}
        if h > 0:
            ins.append(new_edge)
            in_specs.append(pl.BlockSpec(memory_space=pltpu.HBM))
            body = _edge_body_alias
            io_aliases = {7: 1}
        u_h, new_edge = pl.pallas_call(
            body,
            grid=(blks_h,),
            in_specs=in_specs,
            out_specs=[pl.BlockSpec((_BE, D), _rowblk),
                       pl.BlockSpec((_BE, D), _ef_blk)],
            out_shape=[jax.ShapeDtypeStruct((E_PADH, D), f32),
                       jax.ShapeDtypeStruct((E, D), f32)],
            input_output_aliases=io_aliases,
        )(*ins)
        parts.append(scatter(u_h, ridx[h * ROWS_H:(h + 1) * ROWS_H]))

    pslices = [p[c, :N] for p in parts for c in range(NC)]

    new_node = pl.pallas_call(
        _node_body,
        grid=(N // _BNM,),
        in_specs=[pl.BlockSpec((_BNM, D), _rowblk)] * 5 +
                 [pl.BlockSpec((D, H), _full),
                  pl.BlockSpec((D, H), _full),
                  pl.BlockSpec((H, D), _full),
                  pl.BlockSpec((1, H), _full),
                  pl.BlockSpec((1, D), _full)],
        out_specs=pl.BlockSpec((_BNM, D), _rowblk),
        out_shape=jax.ShapeDtypeStruct((N, D), f32),
    )(node_features, *pslices, Wn1a, Wn1b, Wn2, b_n1, b_n2)

    return new_node, new_edge


# speedup vs baseline: 1.1293x; 1.1293x over previous
import functools

import jax
import jax.numpy as jnp
from jax import lax
from jax.experimental import pallas as pl
from jax.experimental.pallas import tpu as pltpu
from jax.experimental.pallas import tpu_sc as plsc

N, E, D, H = 10000, 320000, 128, 128
NC, NS = 2, 16
NW = NC * NS
CHUNK = 128
ROWS2 = 2560
E_PAD = ROWS2 * CHUNK
HK = 2
ROWS_H = ROWS2 // HK
E_PADH = ROWS_H * CHUNK
E_H = E // HK
CPW = ROWS_H // NW
PAIRS = CPW // 4
HPAIRS = CPW // 2
N_PAD = 10240
NPT = N_PAD // NS

_mesh = functools.partial(
    plsc.VectorSubcoreMesh,
    core_axis_name="c", subcore_axis_name="s", num_cores=NC, num_subcores=NS,
)


def _wid():
    return (lax.axis_index("s") * NC + lax.axis_index("c")).astype(jnp.int32)


def _gather_body(ps_hbm, pr_hbm, sidx_hbm, ridx_hbm, g_hbm,
                 sidx_v, ridx_v, sb0, sb1, rb0, rb1,
                 gsem0, gsem1, osem0, osem1):
    wid = _wid()
    base_c = wid * CPW
    pltpu.sync_copy(sidx_hbm.at[pl.ds(base_c, CPW)], sidx_v)
    pltpu.sync_copy(ridx_hbm.at[pl.ds(base_c, CPW)], ridx_v)

    def pair(p, carry):
        c0 = p * 2
        bufs = ((sb0, rb0, gsem0, osem0, c0), (sb1, rb1, gsem1, osem1, c0 + 1))
        for sb, rb, gsem, osem, cb in bufs:
            dst = g_hbm.at[pl.ds((base_c + cb) * CHUNK, CHUNK)]

            @pl.when(p > 0)
            def _(sb=sb, osem=osem, dst=dst):
                pltpu.make_async_copy(dst, sb, osem).wait()

            pltpu.async_copy(ps_hbm.at[sidx_v.at[cb]], sb, gsem)
            pltpu.async_copy(pr_hbm.at[ridx_v.at[cb]], rb, gsem)
        for sb, rb, gsem, osem, cb in bufs:
            dst = g_hbm.at[pl.ds((base_c + cb) * CHUNK, CHUNK)]
            pltpu.make_async_copy(ps_hbm.at[sidx_v.at[cb]], sb, gsem).wait()
            pltpu.make_async_copy(pr_hbm.at[ridx_v.at[cb]], rb, gsem).wait()

            def addrow(i, carry2, sb=sb, rb=rb):
                for l in range(D // 16):
                    sl = pl.ds(l * 16, 16)
                    sb[i, sl] = sb[i, sl] + rb[i, sl]
                return carry2

            lax.fori_loop(0, CHUNK, addrow, 0)
            pltpu.async_copy(sb, dst, osem)
        return carry

    lax.fori_loop(0, CPW // 2, pair, 0)
    pltpu.make_async_copy(g_hbm.at[pl.ds(0, CHUNK)], sb0, osem0).wait()
    pltpu.make_async_copy(g_hbm.at[pl.ds(0, CHUNK)], sb1, osem1).wait()


def _scatter_body(u_hbm, ridx_hbm, out_hbm, idx_v, ub0, ub1,
                  agg_sh, lsem0, lsem1):
    c = lax.axis_index("c").astype(jnp.int32)
    s = lax.axis_index("s").astype(jnp.int32)
    wid = s * NC + c
    zero = jnp.zeros((16,), jnp.float32)
    zb = 32
    for i in range(zb):
        for l in range(D // 16):
            ub0[i, pl.ds(l * 16, 16)] = zero
    zsrc = ub0.at[pl.ds(0, zb)]
    for k in range(NPT // zb):
        pltpu.sync_copy(zsrc, agg_sh.at[pl.ds(s * NPT + k * zb, zb)])
    plsc.subcore_barrier()
    base_c = wid * CPW
    pltpu.sync_copy(ridx_hbm.at[pl.ds(base_c, CPW)], idx_v)
    pltpu.async_copy(u_hbm.at[pl.ds(base_c * CHUNK, CHUNK)], ub0, lsem0)
    pltpu.async_copy(u_hbm.at[pl.ds((base_c + 1) * CHUNK, CHUNK)], ub1, lsem1)

    def pair(p, carry):
        for off, buf, lsem in ((0, ub0, lsem0), (1, ub1, lsem1)):
            cb = 2 * p + off
            src = u_hbm.at[pl.ds((base_c + cb) * CHUNK, CHUNK)]
            pltpu.make_async_copy(src, buf, lsem).wait()
            pltpu.sync_copy(buf, agg_sh.at[idx_v.at[cb]], add=True)

            @pl.when(p < HPAIRS - 1)
            def _(buf=buf, lsem=lsem, cb=cb):
                pltpu.async_copy(
                    u_hbm.at[pl.ds((base_c + cb + 2) * CHUNK, CHUNK)],
                    buf, lsem)
        return carry

    lax.fori_loop(0, HPAIRS, pair, 0)
    plsc.subcore_barrier()
    ob = ub0.shape[0]
    for k in range(NPT // ob):
        off = s * NPT + k * ob
        pltpu.sync_copy(agg_sh.at[pl.ds(off, ob)], ub0)
        pltpu.sync_copy(ub0, out_hbm.at[c].at[pl.ds(off, ob)])


def _proj_body(node_ref, ws_ref, wr_ref, ps_ref, pr_ref):
    x = node_ref[...]
    ps_ref[...] = jnp.dot(x, ws_ref[...], preferred_element_type=jnp.float32)
    pr_ref[...] = jnp.dot(x, wr_ref[...], preferred_element_type=jnp.float32)


def _edge_body(g_ref, ef_ref, we_ref, we2_ref, b1_ref, b2_ref,
               u_ref, ne_ref):
    ef = ef_ref[...]
    x = g_ref[...] + b1_ref[...]
    x = x + jnp.dot(ef, we_ref[...], preferred_element_type=jnp.float32)
    h = jnp.maximum(x, 0.0)
    u = jnp.dot(h, we2_ref[...], preferred_element_type=jnp.float32) + b2_ref[...]
    u_ref[...] = u
    ne_ref[...] = u + ef


def _edge_body_alias(g_ref, ef_ref, we_ref, we2_ref, b1_ref, b2_ref,
                     ne_prev_ref, u_ref, ne_ref):
    del ne_prev_ref
    _edge_body(g_ref, ef_ref, we_ref, we2_ref, b1_ref, b2_ref, u_ref, ne_ref)


def _node_body(nf_ref, p0_ref, p1_ref, p2_ref, p3_ref,
               w1a_ref, w1b_ref, w2_ref, b1_ref, b2_ref, out_ref):
    nf = nf_ref[...]
    agg = (p0_ref[...] + p1_ref[...]) + (p2_ref[...] + p3_ref[...])
    x = (jnp.dot(nf, w1a_ref[...], preferred_element_type=jnp.float32)
         + jnp.dot(agg, w1b_ref[...], preferred_element_type=jnp.float32)
         + b1_ref[...])
    h = jnp.maximum(x, 0.0)
    out_ref[...] = (jnp.dot(h, w2_ref[...], preferred_element_type=jnp.float32)
                    + b2_ref[...] + nf)


_BN = 1024
_BNM = 1000
_BE = 1280


def _full(i):
    return (0, 0)


def _rowblk(i):
    return (i, 0)


def kernel(node_features, edge_features, senders, receivers,
           We1, be1, We2, be2, Wn1, bn1, Wn2, bn2):
    f32 = jnp.float32
    pad_e = E_PAD - E
    ar = jnp.arange(pad_e, dtype=jnp.int32)
    sidx = jnp.concatenate([senders.astype(jnp.int32),
                            ar % N_PAD]).reshape(ROWS2, CHUNK)
    ridx = jnp.concatenate([receivers.astype(jnp.int32),
                            N + ar % (N_PAD - N)]).reshape(ROWS2, CHUNK)
    Ws, Wr, We = We1[:D], We1[D:2 * D], We1[2 * D:]
    Wn1a, Wn1b = Wn1[:D], Wn1[D:]
    b_e1, b_e2 = be1.reshape(1, H), be2.reshape(1, D)
    b_n1, b_n2 = bn1.reshape(1, H), bn2.reshape(1, D)

    ps, pr = pl.pallas_call(
        _proj_body,
        grid=(N_PAD // _BN,),
        in_specs=[pl.BlockSpec((_BN, D), _rowblk),
                  pl.BlockSpec((D, D), _full),
                  pl.BlockSpec((D, D), _full)],
        out_specs=[pl.BlockSpec((_BN, D), _rowblk)] * 2,
        out_shape=[jax.ShapeDtypeStruct((N_PAD, D), f32)] * 2,
    )(node_features, Ws, Wr)

    gather = pl.kernel(
        _gather_body,
        out_type=jax.ShapeDtypeStruct((E_PADH, D), f32),
        mesh=_mesh(),
        scratch_types=[pltpu.VMEM((CPW, CHUNK), jnp.int32),
                       pltpu.VMEM((CPW, CHUNK), jnp.int32),
                       pltpu.VMEM((CHUNK, D), f32),
                       pltpu.VMEM((CHUNK, D), f32),
                       pltpu.VMEM((CHUNK, D), f32),
                       pltpu.VMEM((CHUNK, D), f32),
                       pltpu.SemaphoreType.DMA,
                       pltpu.SemaphoreType.DMA,
                       pltpu.SemaphoreType.DMA,
                       pltpu.SemaphoreType.DMA],
    )
    scatter = pl.kernel(
        _scatter_body,
        out_type=jax.ShapeDtypeStruct((NC, N_PAD, D), f32),
        mesh=_mesh(),
        scratch_types=[pltpu.VMEM((CPW, CHUNK), jnp.int32),
                       pltpu.VMEM((CHUNK, D), f32),
                       pltpu.VMEM((CHUNK, D), f32),
                       pltpu.VMEM_SHARED((N_PAD, D), f32),
                       pltpu.SemaphoreType.DMA,
                       pltpu.SemaphoreType.DMA],
    )

    real_rows = [E_PADH] * (HK - 1) + [E - (HK - 1) * E_PADH]
    blk_off = [h * (E_PADH // _BE) for h in range(HK)]
    new_edge = None
    parts = []
    gpairs = [gather(ps, pr, sidx[h * ROWS_H:(h + 1) * ROWS_H],
                     ridx[h * ROWS_H:(h + 1) * ROWS_H]) for h in range(HK)]
    for h in range(HK):
        g_h = gpairs[h]
        blks_h = real_rows[h] // _BE

        def _ef_blk(i, h=h):
            return (i + blk_off[h], 0)

        ins = [g_h, edge_features, We, We2, b_e1, b_e2]
        in_specs = [pl.BlockSpec((_BE, D), _rowblk),
                    pl.BlockSpec((_BE, D), _ef_blk),
                    pl.BlockSpec((D, H), _full),
                    pl.BlockSpec((H, D), _full),
                    pl.BlockSpec((1, H), _full),
                    pl.BlockSpec((1, D), _full)]
        body = _edge_body
        io_aliases = {}
        if h > 0:
            ins.append(new_edge)
            in_specs.append(pl.BlockSpec(memory_space=pltpu.HBM))
            body = _edge_body_alias
            io_aliases = {6: 1}
        u_h, new_edge = pl.pallas_call(
            body,
            grid=(blks_h,),
            in_specs=in_specs,
            out_specs=[pl.BlockSpec((_BE, D), _rowblk),
                       pl.BlockSpec((_BE, D), _ef_blk)],
            out_shape=[jax.ShapeDtypeStruct((E_PADH, D), f32),
                       jax.ShapeDtypeStruct((E, D), f32)],
            input_output_aliases=io_aliases,
        )(*ins)
        parts.append(scatter(u_h, ridx[h * ROWS_H:(h + 1) * ROWS_H]))

    pslices = [p[c, :N] for p in parts for c in range(NC)]

    new_node = pl.pallas_call(
        _node_body,
        grid=(N // _BNM,),
        in_specs=[pl.BlockSpec((_BNM, D), _rowblk)] * 5 +
                 [pl.BlockSpec((D, H), _full),
                  pl.BlockSpec((D, H), _full),
                  pl.BlockSpec((H, D), _full),
                  pl.BlockSpec((1, H), _full),
                  pl.BlockSpec((1, D), _full)],
        out_specs=pl.BlockSpec((_BNM, D), _rowblk),
        out_shape=jax.ShapeDtypeStruct((N, D), f32),
    )(node_features, *pslices, Wn1a, Wn1b, Wn2, b_n1, b_n2)

    return new_node, new_edge
